# Initial kernel scaffold; baseline (speedup 1.0000x reference)
import jax
import jax.numpy as jnp
from jax.experimental import pallas as pl


def kernel(x, edge_index, W1, b1, W2, b2, We, be, Wc1, bc1, Wc2, bc2):
    raise NotImplementedError("write your pallas kernel here")


# speedup vs baseline: 6.2209x; 6.2209x over previous
import functools

import jax
import jax.numpy as jnp
from jax import lax
from jax.experimental import pallas as pl
from jax.experimental.pallas import tpu as pltpu
from jax.experimental.pallas import tpu_sc as plsc

NC = 2
NS = 16
NW = NC * NS
CH = 80
DEGW = 16

_SC_PARAMS = pltpu.CompilerParams(use_tc_tiling_on_sc=False)


def _worker(c, s, e):
    epw = e // NW
    return c * (e // NC) + s * epw, epw


def _mesh():
    return plsc.VectorSubcoreMesh(core_axis_name="c", subcore_axis_name="s")


def _rpt(n):
    return (n // NS + 7) // 8 * 8


def _deg_kernel(n, e):
    epw = e // NW
    nch = epw // CH
    rpt = _rpt(n)
    npad = NS * rpt

    @functools.partial(
        pl.kernel,
        out_type=jax.ShapeDtypeStruct((NC, npad, DEGW), jnp.float32),
        mesh=_mesh(),
        compiler_params=_SC_PARAMS,
        scratch_types=[
            pltpu.VMEM((CH,), jnp.int32),
            pltpu.VMEM((CH, DEGW), jnp.float32),
            pltpu.VMEM((rpt, DEGW), jnp.float32),
            pltpu.VMEM_SHARED((npad, DEGW), jnp.float32),
        ],
    )
    def k(dst, out, didx, ones, zbuf, acc):
        c = lax.axis_index("c")
        s = lax.axis_index("s")
        base, _ = _worker(c, s, e)

        def initz(i, _):
            zbuf[i, :] = jnp.zeros((16,), jnp.float32)
            return 0

        lax.fori_loop(0, rpt, initz, 0)

        def inito(i, _):
            ones[i, :] = jnp.ones((16,), jnp.float32)
            return 0

        lax.fori_loop(0, CH, inito, 0)
        pltpu.sync_copy(zbuf, acc.at[pl.ds(s * rpt, rpt)])
        plsc.subcore_barrier()

        def body(i, _):
            pltpu.sync_copy(dst.at[pl.ds(base + i * CH, CH)], didx)
            pltpu.sync_copy(ones, acc.at[didx], add=True)
            return 0

        lax.fori_loop(0, nch, body, 0)
        plsc.subcore_barrier()
        pltpu.sync_copy(acc.at[pl.ds(s * rpt, rpt)],
                        out.at[c, pl.ds(s * rpt, rpt)])

    return k


def _seg_kernel(n, e, dh):
    epw = e // NW
    nch = epw // CH
    rpt = _rpt(n)
    npad = NS * rpt

    @functools.partial(
        pl.kernel,
        out_type=[jax.ShapeDtypeStruct((NC, npad, dh), jnp.float32),
                  jax.ShapeDtypeStruct((NC, npad, dh), jnp.float32)],
        mesh=_mesh(),
        compiler_params=_SC_PARAMS,
        scratch_types=[
            pltpu.VMEM((CH,), jnp.int32),
            pltpu.VMEM((CH,), jnp.int32),
            pltpu.VMEM((CH, dh), jnp.float32),
            pltpu.VMEM((rpt, dh), jnp.float32),
            pltpu.VMEM_SHARED((npad, dh), jnp.float32),
            pltpu.SemaphoreType.DMA,
        ],
    )
    def k(ga, gb, src, dst, outa, outb, sidx, didx, rows, zbuf, acc, sem):
        c = lax.axis_index("c")
        s = lax.axis_index("s")
        base, _ = _worker(c, s, e)

        def initz(i, _):
            for j in range(dh // 16):
                zbuf[i, pl.ds(j * 16, 16)] = jnp.zeros((16,), jnp.float32)
            return 0

        lax.fori_loop(0, rpt, initz, 0)

        for g, out in ((ga, outa), (gb, outb)):
            pltpu.sync_copy(zbuf, acc.at[pl.ds(s * rpt, rpt)])
            plsc.subcore_barrier()

            def body(i, _, g=g):
                b = base + i * CH
                pltpu.sync_copy(src.at[pl.ds(b, CH)], sidx)
                pltpu.sync_copy(dst.at[pl.ds(b, CH)], didx)
                pltpu.async_copy(g.at[sidx], rows, sem).wait()
                pltpu.sync_copy(rows, acc.at[didx], add=True)
                return 0

            lax.fori_loop(0, nch, body, 0)
            plsc.subcore_barrier()
            pltpu.sync_copy(acc.at[pl.ds(s * rpt, rpt)],
                            out.at[c, pl.ds(s * rpt, rpt)])
            plsc.subcore_barrier()

    return k


def _gat_kernel(n, e, d):
    epw = e // NW
    nch = epw // CH

    @functools.partial(
        pl.kernel,
        out_type=jax.ShapeDtypeStruct((e, d), jnp.float32),
        mesh=_mesh(),
        compiler_params=_SC_PARAMS,
        scratch_types=[
            pltpu.VMEM((CH,), jnp.int32),
            pltpu.VMEM((CH,), jnp.int32),
            pltpu.VMEM((CH, d), jnp.float32),
            pltpu.VMEM((CH, d), jnp.float32),
            pltpu.SemaphoreType.DMA,
            pltpu.SemaphoreType.DMA,
        ],
    )
    def k(a, bmat, src, dst, out, ridx, cidx, av, bv, sem1, sem2):
        c = lax.axis_index("c")
        s = lax.axis_index("s")
        base, _ = _worker(c, s, e)

        def body(i, _):
            b = base + i * CH
            pltpu.sync_copy(src.at[pl.ds(b, CH)], ridx)
            pltpu.sync_copy(dst.at[pl.ds(b, CH)], cidx)
            cp1 = pltpu.async_copy(a.at[ridx], av, sem1)
            cp2 = pltpu.async_copy(bmat.at[cidx], bv, sem2)
            cp1.wait()
            cp2.wait()

            def comb(r, _):
                for j in range(d // 16):
                    sl = pl.ds(j * 16, 16)
                    av[r, sl] = jnp.maximum(av[r, sl] + bv[r, sl], 0.0)
                return 0

            lax.fori_loop(0, CH, comb, 0)
            pltpu.sync_copy(av, out.at[pl.ds(b, CH)])
            return 0

        lax.fori_loop(0, nch, body, 0)

    return k


def _dinv_of(degp):
    return lax.rsqrt(1.0 + degp[0, :, 0:1] + degp[1, :, 0:1])


def _k1_body(x_ref, w_ref, degp_ref, oa_ref, ob_ref):
    dh = oa_ref.shape[1]
    dinv = _dinv_of(degp_ref[...])
    t = jnp.dot(x_ref[...], w_ref[...],
                preferred_element_type=jnp.float32) * dinv
    oa_ref[...] = t[:, :dh]
    ob_ref[...] = t[:, dh:]


def _k2_body(sa_ref, sb_ref, ga_ref, gb_ref, degp_ref, b1_ref, w2_ref,
             oa_ref, ob_ref):
    dh = oa_ref.shape[1]
    dinv = _dinv_of(degp_ref[...])
    sa = sa_ref[...]
    sb = sb_ref[...]
    b1 = b1_ref[...]
    h1a = jnp.maximum(dinv * (sa[0] + sa[1] + ga_ref[...]) + b1[:, :dh], 0.0)
    h1b = jnp.maximum(dinv * (sb[0] + sb[1] + gb_ref[...]) + b1[:, dh:], 0.0)
    w2 = w2_ref[...]
    t = (jnp.dot(h1a, w2[:dh], preferred_element_type=jnp.float32)
         + jnp.dot(h1b, w2[dh:], preferred_element_type=jnp.float32)) * dinv
    oa_ref[...] = t[:, :dh]
    ob_ref[...] = t[:, dh:]


def _k3_body(sa_ref, sb_ref, ga_ref, gb_ref, degp_ref, b2_ref, we_ref,
             be_ref, a_ref, b_ref):
    dh = ga_ref.shape[1]
    d = 2 * dh
    dinv = _dinv_of(degp_ref[...])
    sa = sa_ref[...]
    sb = sb_ref[...]
    b2 = b2_ref[...]
    h2a = dinv * (sa[0] + sa[1] + ga_ref[...]) + b2[:, :dh]
    h2b = dinv * (sb[0] + sb[1] + gb_ref[...]) + b2[:, dh:]
    we = we_ref[...]
    a_ref[...] = (jnp.dot(h2a, we[:dh], preferred_element_type=jnp.float32)
                  + jnp.dot(h2b, we[dh:d], preferred_element_type=jnp.float32)
                  + be_ref[...])
    b_ref[...] = (jnp.dot(h2a, we[d:d + dh],
                          preferred_element_type=jnp.float32)
                  + jnp.dot(h2b, we[d + dh:],
                            preferred_element_type=jnp.float32))


def _k4_body(f_ref, wc1_ref, bc1_ref, wc2_ref, bc2_ref, o_ref):
    hc = jnp.maximum(
        jnp.dot(f_ref[...], wc1_ref[...],
                preferred_element_type=jnp.float32) + bc1_ref[...], 0.0)
    wdiff = wc2_ref[:, 1:2] - wc2_ref[:, 0:1]
    z = jnp.dot(hc, wdiff, preferred_element_type=jnp.float32) \
        + (bc2_ref[:, 1:2] - bc2_ref[:, 0:1])
    o_ref[...] = jax.nn.sigmoid(z)


def _row_spec(rb, cols):
    return pl.BlockSpec((rb, cols), lambda i: (i, 0))


def _full_spec(shape):
    return pl.BlockSpec(shape, lambda i: tuple(0 for _ in shape))


def kernel(x, edge_index, W1, b1, W2, b2, We, be, Wc1, bc1, Wc2, bc2):
    n, din = x.shape
    e = edge_index.shape[1]
    d = W1.shape[1]
    dh = d // 2
    hdim = Wc1.shape[1]
    assert e % (NW * CH) == 0 and n % NS == 0

    rb = 1000
    degp_spec = pl.BlockSpec((NC, rb, DEGW), lambda i: (0, i, 0))
    seg_spec = pl.BlockSpec((NC, rb, dh), lambda i: (0, i, 0))
    half_spec = _row_spec(rb, dh)

    esrc = edge_index[0]
    edst = edge_index[1]
    degp = _deg_kernel(n, e)(edst)

    g1a, g1b = pl.pallas_call(
        _k1_body,
        grid=(n // rb,),
        in_specs=[_row_spec(rb, din), _full_spec((din, d)), degp_spec],
        out_specs=[half_spec, half_spec],
        out_shape=[jax.ShapeDtypeStruct((n, dh), jnp.float32)] * 2,
    )(x, W1, degp)

    seg = _seg_kernel(n, e, dh)
    s1a, s1b = seg(g1a, g1b, esrc, edst)

    g2a, g2b = pl.pallas_call(
        _k2_body,
        grid=(n // rb,),
        in_specs=[seg_spec, seg_spec, half_spec, half_spec, degp_spec,
                  _full_spec((1, d)), _full_spec((d, d))],
        out_specs=[half_spec, half_spec],
        out_shape=[jax.ShapeDtypeStruct((n, dh), jnp.float32)] * 2,
    )(s1a, s1b, g1a, g1b, degp, b1.reshape(1, d), W2)

    s2a, s2b = seg(g2a, g2b, esrc, edst)

    A, B = pl.pallas_call(
        _k3_body,
        grid=(n // rb,),
        in_specs=[seg_spec, seg_spec, half_spec, half_spec, degp_spec,
                  _full_spec((1, d)), _full_spec((2 * d, d)),
                  _full_spec((1, d))],
        out_specs=[_row_spec(rb, d), _row_spec(rb, d)],
        out_shape=[jax.ShapeDtypeStruct((n, d), jnp.float32)] * 2,
    )(s2a, s2b, g2a, g2b, degp, b2.reshape(1, d), We, be.reshape(1, d))

    F = _gat_kernel(n, e, d)(A, B, esrc, edst)

    eb = 6400
    out2d = pl.pallas_call(
        _k4_body,
        grid=(e // eb,),
        in_specs=[_row_spec(eb, d), _full_spec((d, hdim)),
                  _full_spec((1, hdim)), _full_spec((hdim, 2)),
                  _full_spec((1, 2))],
        out_specs=_row_spec(eb, 1),
        out_shape=jax.ShapeDtypeStruct((e, 1), jnp.float32),
    )(F, Wc1, bc1.reshape(1, hdim), Wc2, bc2.reshape(1, 2))

    return out2d[:, 0]
